# baseline (speedup 1.0000x reference)
import functools

import jax
import jax.numpy as jnp
from jax import lax
from jax.experimental import pallas as pl
from jax.experimental.pallas import tpu as pltpu
from jax.experimental.pallas import tpu_sc as plsc

D = 64
BBLK = 128
DT = D // 8
NBUF = 8


@functools.lru_cache(maxsize=None)
def _make(S: int, B: int):
    info = plsc.get_sparse_core_info()
    nc = info.num_cores
    nw = nc * info.num_subcores
    assert B == nw * BBLK and S % NBUF == 0
    ngroups = S // NBUF
    mesh = plsc.VectorSubcoreMesh(core_axis_name="c", subcore_axis_name="s")

    @functools.partial(
        pl.kernel,
        mesh=mesh,
        out_type=jax.ShapeDtypeStruct((S, DT, nw, 8, BBLK), jnp.float32),
        scratch_types=[
            pltpu.VMEM((S, BBLK), jnp.int32),
            [pltpu.VMEM((BBLK, D), jnp.float32) for _ in range(NBUF)],
            [pltpu.VMEM((DT, 8, BBLK + 1), jnp.float32) for _ in range(2)],
            [pltpu.SemaphoreType.DMA for _ in range(NBUF)],
            [pltpu.SemaphoreType.DMA for _ in range(2)],
        ],
        compiler_params=pltpu.CompilerParams(
            use_tc_tiling_on_sc=False, needs_layout_passes=False
        ),
    )
    def body(idsT, tableT, out5, ids_v, gbufs, tbufs, gsems, tsems):
        w = lax.axis_index("s") * nc + lax.axis_index("c")
        pltpu.sync_copy(idsT.at[:, pl.ds(w * BBLK, BBLK)], ids_v)

        def start(s, b):
            pltpu.async_copy(tableT.at[ids_v.at[s]], gbufs[b], gsems[b])

        def wait(s, b):
            pltpu.make_async_copy(tableT.at[ids_v.at[s]], gbufs[b], gsems[b]).wait()

        for b in range(NBUF):
            start(b, b)

        iota = lax.iota(jnp.int32, 16)
        nk = D // 16
        dt_idx = [(16 * k + iota) // 8 for k in range(nk)]
        dd_idx = [(16 * k + iota) % 8 for k in range(nk)]

        def group(j, carry):
            for b in range(NBUF):
                s = j * NBUF + b
                tb = b % 2
                wait(s, b)

                if b >= 2:
                    pltpu.make_async_copy(
                        tbufs[tb].at[:, :, pl.ds(0, BBLK)],
                        out5.at[s - 2, :, w],
                        tsems[tb],
                    ).wait()
                else:

                    @pl.when(j > 0)
                    def _():
                        pltpu.make_async_copy(
                            tbufs[tb].at[:, :, pl.ds(0, BBLK)],
                            out5.at[s - 2, :, w],
                            tsems[tb],
                        ).wait()

                def tline(tk, carry2):
                    col = lax.broadcast(tk, (16,))
                    vs = [gbufs[b][tk, pl.ds(16 * k, 16)] for k in range(nk)]
                    for k in range(nk):
                        plsc.store_scatter(
                            tbufs[tb], [dt_idx[k], dd_idx[k], col], vs[k]
                        )
                    return carry2

                lax.fori_loop(0, BBLK, tline, 0)

                pltpu.async_copy(
                    tbufs[tb].at[:, :, pl.ds(0, BBLK)], out5.at[s, :, w], tsems[tb]
                )

                @pl.when(j < ngroups - 1)
                def _():
                    start(s + NBUF, b)

            return carry

        lax.fori_loop(0, ngroups, group, 0)

        pltpu.make_async_copy(
            tbufs[0].at[:, :, pl.ds(0, BBLK)], out5.at[S - 2, :, w], tsems[0]
        ).wait()
        pltpu.make_async_copy(
            tbufs[1].at[:, :, pl.ds(0, BBLK)], out5.at[S - 1, :, w], tsems[1]
        ).wait()

    return body


@functools.lru_cache(maxsize=None)
def _make_detile(V: int):
    info = plsc.get_sparse_core_info()
    nc = info.num_cores
    nw = nc * info.num_subcores
    nt = V // BBLK
    tail = V - nt * BBLK
    tpw = nt // nw
    assert tpw % 4 == 0 and nt % nw == 4 and tail == 64
    mesh = plsc.VectorSubcoreMesh(core_axis_name="c", subcore_axis_name="s")

    @functools.partial(
        pl.kernel,
        mesh=mesh,
        out_type=jax.ShapeDtypeStruct((V // 2, BBLK), jnp.float32),
        scratch_types=[
            [pltpu.VMEM((D, BBLK), jnp.float32) for _ in range(4)],
            [pltpu.VMEM((D, BBLK), jnp.float32) for _ in range(2)],
            pltpu.VMEM((D, 64), jnp.float32),
            [pltpu.SemaphoreType.DMA for _ in range(4)],
            [pltpu.SemaphoreType.DMA for _ in range(2)],
        ],
        compiler_params=pltpu.CompilerParams(needs_layout_passes=False),
    )
    def body(tT, tail_t, z, ins, t2s, vb, isems, osems):
        w = lax.axis_index("s") * nc + lax.axis_index("c")
        iota = lax.iota(jnp.int32, 16)
        perm = [(iota + m) % 16 for m in range(16)]
        ih = iota // 2
        par = (iota % 2) * 64

        def tile_of(i):
            return w + nw * i

        def start_in(i, b):
            pltpu.async_copy(
                tT.at[:, pl.ds(tile_of(i) * BBLK, BBLK)], ins[b], isems[b]
            )

        def wait_in(i, b):
            pltpu.make_async_copy(
                tT.at[:, pl.ds(tile_of(i) * BBLK, BBLK)], ins[b], isems[b]
            ).wait()

        def start_out(i, p):
            pltpu.async_copy(t2s[p], z.at[pl.ds(64 * tile_of(i), 64)], osems[p])

        def wait_out(i, p):
            pltpu.make_async_copy(
                t2s[p], z.at[pl.ds(64 * tile_of(i), 64)], osems[p]
            ).wait()

        def transpose_tile(src, p, nv):
            def vblock(blk, carry):
                v0 = blk * 16
                vcol = v0 + iota
                zrow = v0 // 2 + ih
                for dk in range(4):
                    d0 = dk * 16
                    rowvs = [d0 + perm[m] for m in range(16)]
                    for mh in range(2):
                        vvs = [
                            plsc.load_gather(src, [rowvs[8 * mh + m], vcol])
                            for m in range(8)
                        ]
                        for m in range(8):
                            plsc.store_scatter(
                                t2s[p],
                                [zrow, par + rowvs[8 * mh + m]],
                                vvs[m],
                            )
                return carry

            lax.fori_loop(0, nv, vblock, 0)

        for b in range(4):
            start_in(b, b)

        def group(j, carry):
            for b in range(4):
                i = j * 4 + b
                p = b % 2
                wait_in(i, b)
                if b >= 2:
                    wait_out(i - 2, p)
                else:

                    @pl.when(j > 0)
                    def _():
                        wait_out(i - 2, p)

                transpose_tile(ins[b], p, 8)
                start_out(i, p)

                @pl.when(j < tpw // 4 - 1)
                def _():
                    start_in(i + 4, b)

            return carry

        lax.fori_loop(0, tpw // 4, group, 0)
        wait_out(tpw - 2, 0)
        wait_out(tpw - 1, 1)

        @pl.when(w < nt % nw)
        def _():
            pltpu.sync_copy(tT.at[:, pl.ds(tile_of(tpw) * BBLK, BBLK)], ins[0])
            transpose_tile(ins[0], 0, 8)
            pltpu.sync_copy(t2s[0], z.at[pl.ds(64 * tile_of(tpw), 64)])

        @pl.when(w == nt % nw)
        def _():
            pltpu.sync_copy(tail_t, vb)
            transpose_tile(vb, 0, 4)
            pltpu.sync_copy(
                t2s[0].at[pl.ds(0, 32)], z.at[pl.ds(64 * nt, 32)]
            )

    return body


def kernel(input_ids, table):
    b, s = input_ids.shape
    d = table.shape[1]
    idsT = input_ids.T.astype(jnp.int32)
    v = table.shape[0]
    tail_t = table[(v // BBLK) * BBLK :].T
    z = _make_detile(v)(table.T, tail_t)
    out5 = _make(s, b)(idsT, z.reshape(v, d))
    return out5.transpose(2, 4, 0, 1, 3).reshape(b, s, d)


# speedup vs baseline: 1.1037x; 1.1037x over previous
import functools

import jax
import jax.numpy as jnp
from jax import lax
from jax.experimental import pallas as pl
from jax.experimental.pallas import tpu as pltpu
from jax.experimental.pallas import tpu_sc as plsc

D = 64
BBLK = 128
DT = D // 8
NBUF = 4


@functools.lru_cache(maxsize=None)
def _make(S: int, B: int):
    info = plsc.get_sparse_core_info()
    nc = info.num_cores
    nw = nc * info.num_subcores
    assert B == nw * BBLK and S % NBUF == 0
    ngroups = S // NBUF
    mesh = plsc.VectorSubcoreMesh(core_axis_name="c", subcore_axis_name="s")

    @functools.partial(
        pl.kernel,
        mesh=mesh,
        out_type=jax.ShapeDtypeStruct((S, DT, nw, 8, BBLK), jnp.float32),
        scratch_types=[
            pltpu.VMEM((S, BBLK), jnp.int32),
            [pltpu.VMEM((BBLK, D), jnp.float32) for _ in range(NBUF)],
            [pltpu.VMEM((DT, 8, BBLK), jnp.float32) for _ in range(2)],
            [pltpu.SemaphoreType.DMA for _ in range(NBUF)],
            [pltpu.SemaphoreType.DMA for _ in range(2)],
        ],
        compiler_params=pltpu.CompilerParams(
            use_tc_tiling_on_sc=False, needs_layout_passes=False
        ),
    )
    def body(idsT, tableT, out5, ids_v, gbufs, tbufs, gsems, tsems):
        w = lax.axis_index("s") * nc + lax.axis_index("c")
        pltpu.sync_copy(idsT.at[:, pl.ds(w * BBLK, BBLK)], ids_v)

        def start(s, b):
            pltpu.async_copy(tableT.at[ids_v.at[s]], gbufs[b], gsems[b])

        def wait(s, b):
            pltpu.make_async_copy(tableT.at[ids_v.at[s]], gbufs[b], gsems[b]).wait()

        for b in range(NBUF):
            start(b, b)

        iota = lax.iota(jnp.int32, 16)
        perm = [(iota + m) % 16 for m in range(16)]

        def group(j, carry):
            for b in range(NBUF):
                s = j * NBUF + b
                tb = b % 2
                wait(s, b)

                if b >= 2:
                    pltpu.make_async_copy(
                        tbufs[tb], out5.at[s - 2, :, w], tsems[tb]
                    ).wait()
                else:

                    @pl.when(j > 0)
                    def _():
                        pltpu.make_async_copy(
                            tbufs[tb], out5.at[s - 2, :, w], tsems[tb]
                        ).wait()

                def vblock(blk, carry2):
                    vcol = blk * 16 + iota
                    for dk in range(4):
                        d0 = dk * 16
                        rowvs = [d0 + perm[m] for m in range(16)]
                        dhs = [rowvs[m] // 8 for m in range(16)]
                        dls = [rowvs[m] % 8 for m in range(16)]
                        for mh in range(2):
                            vvs = [
                                plsc.load_gather(
                                    gbufs[b], [vcol, rowvs[8 * mh + m]]
                                )
                                for m in range(8)
                            ]
                            for m in range(8):
                                plsc.store_scatter(
                                    tbufs[tb],
                                    [dhs[8 * mh + m], dls[8 * mh + m], vcol],
                                    vvs[m],
                                )
                    return carry2

                lax.fori_loop(0, BBLK // 16, vblock, 0)

                pltpu.async_copy(tbufs[tb], out5.at[s, :, w], tsems[tb])

                @pl.when(j < ngroups - 1)
                def _():
                    start(s + NBUF, b)

            return carry

        lax.fori_loop(0, ngroups, group, 0)

        pltpu.make_async_copy(tbufs[0], out5.at[S - 2, :, w], tsems[0]).wait()
        pltpu.make_async_copy(tbufs[1], out5.at[S - 1, :, w], tsems[1]).wait()

    return body


@functools.lru_cache(maxsize=None)
def _make_detile(V: int):
    info = plsc.get_sparse_core_info()
    nc = info.num_cores
    nw = nc * info.num_subcores
    nt = V // BBLK
    tail = V - nt * BBLK
    tpw = nt // nw
    assert tpw % 4 == 0 and nt % nw == 4 and tail == 64
    mesh = plsc.VectorSubcoreMesh(core_axis_name="c", subcore_axis_name="s")

    @functools.partial(
        pl.kernel,
        mesh=mesh,
        out_type=jax.ShapeDtypeStruct((V // 2, BBLK), jnp.float32),
        scratch_types=[
            [pltpu.VMEM((D, BBLK), jnp.float32) for _ in range(4)],
            [pltpu.VMEM((D, BBLK), jnp.float32) for _ in range(2)],
            pltpu.VMEM((D, 64), jnp.float32),
            [pltpu.SemaphoreType.DMA for _ in range(4)],
            [pltpu.SemaphoreType.DMA for _ in range(2)],
        ],
        compiler_params=pltpu.CompilerParams(needs_layout_passes=False),
    )
    def body(tT, tail_t, z, ins, t2s, vb, isems, osems):
        w = lax.axis_index("s") * nc + lax.axis_index("c")
        iota = lax.iota(jnp.int32, 16)
        perm = [(iota + m) % 16 for m in range(16)]
        ih = iota // 2
        par = (iota % 2) * 64

        def tile_of(i):
            return w + nw * i

        def start_in(i, b):
            pltpu.async_copy(
                tT.at[:, pl.ds(tile_of(i) * BBLK, BBLK)], ins[b], isems[b]
            )

        def wait_in(i, b):
            pltpu.make_async_copy(
                tT.at[:, pl.ds(tile_of(i) * BBLK, BBLK)], ins[b], isems[b]
            ).wait()

        def start_out(i, p):
            pltpu.async_copy(t2s[p], z.at[pl.ds(64 * tile_of(i), 64)], osems[p])

        def wait_out(i, p):
            pltpu.make_async_copy(
                t2s[p], z.at[pl.ds(64 * tile_of(i), 64)], osems[p]
            ).wait()

        def transpose_tile(src, p, nv):
            def vblock(blk, carry):
                v0 = blk * 16
                vcol = v0 + iota
                zrow = v0 // 2 + ih
                for dk in range(4):
                    d0 = dk * 16
                    rowvs = [d0 + perm[m] for m in range(16)]
                    for mh in range(2):
                        vvs = [
                            plsc.load_gather(src, [rowvs[8 * mh + m], vcol])
                            for m in range(8)
                        ]
                        for m in range(8):
                            plsc.store_scatter(
                                t2s[p],
                                [zrow, par + rowvs[8 * mh + m]],
                                vvs[m],
                            )
                return carry

            lax.fori_loop(0, nv, vblock, 0)

        for b in range(4):
            start_in(b, b)

        def group(j, carry):
            for b in range(4):
                i = j * 4 + b
                p = b % 2
                wait_in(i, b)
                if b >= 2:
                    wait_out(i - 2, p)
                else:

                    @pl.when(j > 0)
                    def _():
                        wait_out(i - 2, p)

                transpose_tile(ins[b], p, 8)
                start_out(i, p)

                @pl.when(j < tpw // 4 - 1)
                def _():
                    start_in(i + 4, b)

            return carry

        lax.fori_loop(0, tpw // 4, group, 0)
        wait_out(tpw - 2, 0)
        wait_out(tpw - 1, 1)

        @pl.when(w < nt % nw)
        def _():
            pltpu.sync_copy(tT.at[:, pl.ds(tile_of(tpw) * BBLK, BBLK)], ins[0])
            transpose_tile(ins[0], 0, 8)
            pltpu.sync_copy(t2s[0], z.at[pl.ds(64 * tile_of(tpw), 64)])

        @pl.when(w == nt % nw)
        def _():
            pltpu.sync_copy(tail_t, vb)
            transpose_tile(vb, 0, 4)
            pltpu.sync_copy(
                t2s[0].at[pl.ds(0, 32)], z.at[pl.ds(64 * nt, 32)]
            )

    return body


def kernel(input_ids, table):
    b, s = input_ids.shape
    d = table.shape[1]
    idsT = input_ids.T.astype(jnp.int32)
    v = table.shape[0]
    tail_t = table[(v // BBLK) * BBLK :].T
    z = _make_detile(v)(table.T, tail_t)
    out5 = _make(s, b)(idsT, z.reshape(v, d))
    return out5.transpose(2, 4, 0, 1, 3).reshape(b, s, d)
